# Initial kernel scaffold; baseline (speedup 1.0000x reference)
import jax
import jax.numpy as jnp
from jax.experimental import pallas as pl


def kernel(X, edge_index, params):
    raise NotImplementedError("write your pallas kernel here")


# speedup vs baseline: 3.3873x; 3.3873x over previous
import functools

import jax
import jax.numpy as jnp
from jax import lax
from jax.experimental import pallas as pl
from jax.experimental.pallas import tpu as pltpu
from jax.experimental.pallas import tpu_sc as plsc

N = 10000
D = 128
E = 320000
NC = 2
NS = 16
NW = NC * NS
CH = 64
STEPS = 160
EP = NW * STEPS * CH
NP = 10240
ZR = NP // NS
BN_EPS = 1e-5
BLK = 1000
GRID = N // BLK

_sc_mesh = plsc.VectorSubcoreMesh(
    core_axis_name="c", subcore_axis_name="s", num_cores=NC, num_subcores=NS)


@functools.partial(
    pl.kernel,
    out_type=jax.ShapeDtypeStruct((NC, NP, D), jnp.float32),
    mesh=_sc_mesh,
    scratch_types=[
        pltpu.VMEM_SHARED((NP, D), jnp.float32),
        pltpu.VMEM((2, CH), jnp.int32),
        pltpu.VMEM((2, CH), jnp.int32),
        pltpu.VMEM((CH, D), jnp.float32),
        pltpu.VMEM((CH, D), jnp.float32),
        pltpu.SemaphoreType.DMA,
        pltpu.SemaphoreType.DMA,
        pltpu.SemaphoreType.DMA,
        pltpu.SemaphoreType.DMA,
    ],
)
def _sc_segsum(r_hbm, ed_hbm, z_hbm, out_hbm,
               acc, ibuf0, ibuf1, rbuf0, rbuf1, isem0, isem1, rsem0, rsem1):
    c = lax.axis_index("c")
    s = lax.axis_index("s")
    w = c * NS + s
    pltpu.sync_copy(z_hbm, acc.at[pl.ds(s * ZR, ZR)])
    pltpu.sync_copy(ed_hbm.at[w, 0], ibuf0)
    pltpu.async_copy(ed_hbm.at[w, 1], ibuf1, isem1)
    plsc.subcore_barrier()
    pltpu.async_copy(r_hbm.at[ibuf0.at[0]], rbuf0, rsem0)

    def body(g, carry):
        for b in (0, 1):
            ib, ibo = (ibuf0, ibuf1) if b == 0 else (ibuf1, ibuf0)
            rb, rbo = (rbuf0, rbuf1) if b == 0 else (rbuf1, rbuf0)
            isem, isemo = (isem0, isem1) if b == 0 else (isem1, isem0)
            rsem, rsemo = (rsem0, rsem1) if b == 0 else (rsem1, rsem0)
            j = g * 2 + b

            @pl.when(j + 1 < STEPS)
            def _():
                pltpu.make_async_copy(ed_hbm.at[w, 0], ibo, isemo).wait()
                pltpu.async_copy(r_hbm.at[ibo.at[0]], rbo, rsemo)

            pltpu.make_async_copy(r_hbm.at[ib.at[0]], rb, rsem).wait()
            pltpu.sync_copy(rb, acc.at[ib.at[1]], add=True)

            @pl.when(j + 2 < STEPS)
            def _():
                pltpu.async_copy(ed_hbm.at[w, j + 2], ib, isem)
        return carry

    lax.fori_loop(0, STEPS // 2, body, 0)
    plsc.subcore_barrier()
    pltpu.sync_copy(acc.at[pl.ds(s * ZR, ZR)], out_hbm.at[c, pl.ds(s * ZR, ZR)])


def _relu_body(x_ref, o_ref):
    o_ref[...] = jnp.maximum(x_ref[...], 0.0)


def _mlp_body(eps_ref, x_ref, p_ref, w1_ref, b1_ref, w2_ref, b2_ref,
              y_ref, st_ref):
    i = pl.program_id(0)
    z = x_ref[...] * eps_ref[0, 0] + p_ref[0] + p_ref[1]
    h = jnp.maximum(
        jnp.dot(z, w1_ref[...], preferred_element_type=jnp.float32)
        + b1_ref[...], 0.0)
    y = (jnp.dot(h, w2_ref[...], preferred_element_type=jnp.float32)
         + b2_ref[...])
    y_ref[...] = y

    @pl.when(i == 0)
    def _():
        st_ref[...] = jnp.zeros_like(st_ref)

    st = jnp.concatenate(
        [jnp.sum(y, axis=0)[None], jnp.sum(y * y, axis=0)[None],
         jnp.zeros((6, D), jnp.float32)], axis=0)
    st_ref[...] += st


def _mlp_res_body(eps_ref, x_ref, p_ref, w1_ref, b1_ref, w2_ref, b2_ref,
                  o_ref):
    z = x_ref[...] * eps_ref[0, 0] + p_ref[0] + p_ref[1]
    h = jnp.maximum(
        jnp.dot(z, w1_ref[...], preferred_element_type=jnp.float32)
        + b1_ref[...], 0.0)
    o_ref[...] = (jnp.dot(h, w2_ref[...], preferred_element_type=jnp.float32)
                  + b2_ref[...] + x_ref[...])


def _bn_body(y_ref, st_ref, g_ref, b_ref, x0_ref, xo_ref, r_ref):
    mean = st_ref[0:1, :] * (1.0 / N)
    var = st_ref[1:2, :] * (1.0 / N) - mean * mean
    scale = lax.rsqrt(var + BN_EPS) * g_ref[...]
    xn = (y_ref[...] - mean) * scale + b_ref[...] + x0_ref[...]
    xo_ref[...] = xn
    r_ref[...] = jnp.maximum(xn, 0.0)


_row_spec = pl.BlockSpec((BLK, D), lambda i: (i, 0))
_full_spec = pl.BlockSpec((D, D), lambda i: (0, 0))
_vec_spec = pl.BlockSpec((1, D), lambda i: (0, 0))
_st_spec = pl.BlockSpec((8, D), lambda i: (0, 0))
_p_spec = pl.BlockSpec((NC, BLK, D), lambda i: (0, i, 0))
_smem_spec = pl.BlockSpec(memory_space=pltpu.SMEM)

_relu_call = pl.pallas_call(
    _relu_body,
    grid=(GRID,),
    in_specs=[_row_spec],
    out_specs=_row_spec,
    out_shape=jax.ShapeDtypeStruct((N, D), jnp.float32),
)

_mlp_call = pl.pallas_call(
    _mlp_body,
    grid=(GRID,),
    in_specs=[_smem_spec, _row_spec, _p_spec, _full_spec, _vec_spec,
              _full_spec, _vec_spec],
    out_specs=[_row_spec, _st_spec],
    out_shape=[jax.ShapeDtypeStruct((N, D), jnp.float32),
               jax.ShapeDtypeStruct((8, D), jnp.float32)],
)

_mlp_res_call = pl.pallas_call(
    _mlp_res_body,
    grid=(GRID,),
    in_specs=[_smem_spec, _row_spec, _p_spec, _full_spec, _vec_spec,
              _full_spec, _vec_spec],
    out_specs=_row_spec,
    out_shape=jax.ShapeDtypeStruct((N, D), jnp.float32),
)

_bn_call = pl.pallas_call(
    _bn_body,
    grid=(GRID,),
    in_specs=[_row_spec, _st_spec, _vec_spec, _vec_spec, _row_spec],
    out_specs=[_row_spec, _row_spec],
    out_shape=[jax.ShapeDtypeStruct((N, D), jnp.float32),
               jax.ShapeDtypeStruct((N, D), jnp.float32)],
)


def kernel(X, edge_index, params):
    src = edge_index[0].astype(jnp.int32)
    dst = edge_index[1].astype(jnp.int32)
    pad = EP - E
    src_p = jnp.concatenate(
        [src, jnp.zeros((pad,), jnp.int32)]).reshape(NW, STEPS, 1, CH)
    dst_p = jnp.concatenate(
        [dst, jnp.full((pad,), N, jnp.int32)]).reshape(NW, STEPS, 1, CH)
    ed_p = jnp.concatenate([src_p, dst_p], axis=2)
    zrows = jnp.zeros((ZR, D), jnp.float32)

    x = X
    R = _relu_call(X)
    for li, p in enumerate(params):
        scale = (1.0 + p['eps']).reshape(1, 1)
        b1 = p['b1'].reshape(1, D)
        b2 = p['b2'].reshape(1, D)
        P = _sc_segsum(R, ed_p, zrows)
        if li < len(params) - 1:
            y, st = _mlp_call(scale, x, P, p['W1'], b1, p['W2'], b2)
            x, R = _bn_call(y, st, p['gamma'].reshape(1, D),
                            p['beta'].reshape(1, D), x)
        else:
            x = _mlp_res_call(scale, x, P, p['W1'], b1, p['W2'], b2)
    return x
